# baseline (device time: 258599 ns/iter reference)
import functools

import jax
import jax.numpy as jnp
from jax import lax
from jax.experimental import pallas as pl
from jax.experimental.pallas import tpu as pltpu


def kernel(x):
    m, n = x.shape
    H = m // 2
    CH = 512
    K = m // CH
    KD = H // CH

    SIZES = [512] * 15 + [256, 128, 64, 32, 32]
    assert sum(SIZES) == H
    OFFS = [sum(SIZES[:i]) for i in range(len(SIZES))]
    NS = len(SIZES)

    def body(x_ref, out_ref, vshard, vload, lsems, ssem,
             ysend, yrecv, xsend, xrecv):
        my_x = lax.axis_index("x")
        my_y = lax.axis_index("y")
        ynbr = (my_x, 1 - my_y)
        xnbr = (1 - my_x, my_y)

        bar = pltpu.get_barrier_semaphore()
        for nbr in (ynbr, xnbr):
            pl.semaphore_signal(bar, inc=1, device_id=nbr,
                                device_id_type=pl.DeviceIdType.MESH)
        pl.semaphore_wait(bar, 2)

        dhalf = my_x * H
        phalf = (1 - my_x) * H

        def lofs(i):
            if i < KD:
                return dhalf + i * CH
            return phalf + (i - KD) * CH

        def start_load(i):
            slot = i % 2
            c = pltpu.make_async_copy(
                x_ref.at[pl.ds(lofs(i), CH)], vload.at[slot], lsems.at[slot])
            c.start()
            return c

        loads = {0: start_load(0)}
        y_rdmas = []
        next_send = 0
        for i in range(K):
            if i + 1 < K:
                loads[i + 1] = start_load(i + 1)
            loads[i].wait()
            vshard[pl.ds(lofs(i), CH), :] = vload[i % 2].astype(jnp.bfloat16)
            if i < KD:
                converted = (i + 1) * CH
                while next_send < NS and OFFS[next_send] + SIZES[next_send] <= converted:
                    o, s, j = OFFS[next_send], SIZES[next_send], next_send
                    r = pltpu.make_async_remote_copy(
                        src_ref=vshard.at[pl.ds(dhalf + o, s)],
                        dst_ref=out_ref.at[pl.ds(my_y * m + dhalf + o, s)],
                        send_sem=ysend.at[j], recv_sem=yrecv.at[j],
                        device_id=ynbr, device_id_type=pl.DeviceIdType.MESH)
                    r.start()
                    y_rdmas.append(r)
                    next_send += 1

        st = pltpu.make_async_copy(
            vshard, out_ref.at[pl.ds(my_y * m, m)], ssem)
        st.start()

        fw_rdmas = []
        for j in range(NS):
            o, s = OFFS[j], SIZES[j]
            y_rdmas[j].wait_recv()
            rofs = (1 - my_y) * m + dhalf + o
            fw = pltpu.make_async_remote_copy(
                src_ref=out_ref.at[pl.ds(rofs, s)],
                dst_ref=out_ref.at[pl.ds(rofs, s)],
                send_sem=xsend.at[j], recv_sem=xrecv.at[j],
                device_id=xnbr, device_id_type=pl.DeviceIdType.MESH)
            fw.start()
            fw_rdmas.append(fw)

        for r in y_rdmas:
            r.wait_send()
        for r in fw_rdmas:
            r.wait_send()
        for r in fw_rdmas:
            r.wait_recv()
        st.wait()

        @functools.partial(pl.run_scoped,
                           second_barrier=pltpu.SemaphoreType.REGULAR)
        def _(second_barrier):
            for nbr in (ynbr, xnbr):
                pl.semaphore_signal(second_barrier, inc=1, device_id=nbr,
                                    device_id_type=pl.DeviceIdType.MESH)
            pl.semaphore_wait(second_barrier, 2)

    return pl.pallas_call(
        body,
        out_shape=jax.ShapeDtypeStruct((2 * m, n), jnp.bfloat16),
        in_specs=[pl.BlockSpec(memory_space=pltpu.MemorySpace.HBM)],
        out_specs=pl.BlockSpec(memory_space=pltpu.MemorySpace.HBM),
        scratch_shapes=[
            pltpu.VMEM((m, n), jnp.bfloat16),
            pltpu.VMEM((2, CH, n), jnp.float32),
            pltpu.SemaphoreType.DMA((2,)),
            pltpu.SemaphoreType.DMA,
            pltpu.SemaphoreType.DMA((NS,)),
            pltpu.SemaphoreType.DMA((NS,)),
            pltpu.SemaphoreType.DMA((NS,)),
            pltpu.SemaphoreType.DMA((NS,)),
        ],
        compiler_params=pltpu.CompilerParams(
            collective_id=0, vmem_limit_bytes=48 * 1024 * 1024),
    )(x)


# device time: 255164 ns/iter; 1.0135x vs baseline; 1.0135x over previous
import functools

import jax
import jax.numpy as jnp
from jax import lax
from jax.experimental import pallas as pl
from jax.experimental.pallas import tpu as pltpu


def kernel(x):
    m, n = x.shape
    H = m // 2
    CH = 512
    K = m // CH
    KD = H // CH
    KP = K - KD

    SIZES = [64, 128, 256, 512] + [1024] * 6 + [512, 256, 128, 128, 64]
    assert sum(SIZES) == H
    OFFS = [sum(SIZES[:i]) for i in range(len(SIZES))]
    NS = len(SIZES)

    def body(x_ref, out_ref, vshard, vload, lsems, ssems,
             ysend, yrecv, xsend, xrecv):
        my_x = lax.axis_index("x")
        my_y = lax.axis_index("y")
        ynbr = (my_x, 1 - my_y)
        xnbr = (1 - my_x, my_y)

        bar = pltpu.get_barrier_semaphore()
        for nbr in (ynbr, xnbr):
            pl.semaphore_signal(bar, inc=1, device_id=nbr,
                                device_id_type=pl.DeviceIdType.MESH)
        pl.semaphore_wait(bar, 2)

        dhalf = my_x * H
        phalf = (1 - my_x) * H

        def lofs(i):
            if i < KD:
                return dhalf + i * CH
            return phalf + (i - KD) * CH

        def start_load(i):
            slot = i % 2
            c = pltpu.make_async_copy(
                x_ref.at[pl.ds(lofs(i), CH)], vload.at[slot], lsems.at[slot])
            c.start()
            return c

        def convert(i, loads):
            if i + 1 < K:
                loads[i + 1] = start_load(i + 1)
            loads[i].wait()
            vshard[pl.ds(lofs(i), CH), :] = vload[i % 2].astype(jnp.bfloat16)

        loads = {0: start_load(0)}
        y_rdmas = []
        next_send = 0
        for i in range(KD):
            convert(i, loads)
            converted = (i + 1) * CH
            while next_send < NS and OFFS[next_send] + SIZES[next_send] <= converted:
                o, s, j = OFFS[next_send], SIZES[next_send], next_send
                r = pltpu.make_async_remote_copy(
                    src_ref=vshard.at[pl.ds(dhalf + o, s)],
                    dst_ref=out_ref.at[pl.ds(my_y * m + dhalf + o, s)],
                    send_sem=ysend.at[j], recv_sem=yrecv.at[j],
                    device_id=ynbr, device_id_type=pl.DeviceIdType.MESH)
                r.start()
                y_rdmas.append(r)
                next_send += 1

        stores = []
        st = pltpu.make_async_copy(
            vshard.at[pl.ds(dhalf, H)],
            out_ref.at[pl.ds(my_y * m + dhalf, H)], ssems.at[0])
        st.start()
        stores.append(st)

        fw_rdmas = []
        next_conv = KD
        for j in range(NS):
            o, s = OFFS[j], SIZES[j]
            y_rdmas[j].wait_recv()
            rofs = (1 - my_y) * m + dhalf + o
            fw = pltpu.make_async_remote_copy(
                src_ref=out_ref.at[pl.ds(rofs, s)],
                dst_ref=out_ref.at[pl.ds(rofs, s)],
                send_sem=xsend.at[j], recv_sem=xrecv.at[j],
                device_id=xnbr, device_id_type=pl.DeviceIdType.MESH)
            fw.start()
            fw_rdmas.append(fw)
            for _ in range(2):
                if next_conv < K:
                    convert(next_conv, loads)
                    p = next_conv - KD
                    sc = pltpu.make_async_copy(
                        vshard.at[pl.ds(phalf + p * CH, CH)],
                        out_ref.at[pl.ds(my_y * m + phalf + p * CH, CH)],
                        ssems.at[1 + p])
                    sc.start()
                    stores.append(sc)
                    next_conv += 1

        for r in y_rdmas:
            r.wait_send()
        for r in fw_rdmas:
            r.wait_send()
        for r in fw_rdmas:
            r.wait_recv()
        for sc in stores:
            sc.wait()

        @functools.partial(pl.run_scoped,
                           second_barrier=pltpu.SemaphoreType.REGULAR)
        def _(second_barrier):
            for nbr in (ynbr, xnbr):
                pl.semaphore_signal(second_barrier, inc=1, device_id=nbr,
                                    device_id_type=pl.DeviceIdType.MESH)
            pl.semaphore_wait(second_barrier, 2)

    return pl.pallas_call(
        body,
        out_shape=jax.ShapeDtypeStruct((2 * m, n), jnp.bfloat16),
        in_specs=[pl.BlockSpec(memory_space=pltpu.MemorySpace.HBM)],
        out_specs=pl.BlockSpec(memory_space=pltpu.MemorySpace.HBM),
        scratch_shapes=[
            pltpu.VMEM((m, n), jnp.bfloat16),
            pltpu.VMEM((2, CH, n), jnp.float32),
            pltpu.SemaphoreType.DMA((2,)),
            pltpu.SemaphoreType.DMA((1 + KP,)),
            pltpu.SemaphoreType.DMA((NS,)),
            pltpu.SemaphoreType.DMA((NS,)),
            pltpu.SemaphoreType.DMA((NS,)),
            pltpu.SemaphoreType.DMA((NS,)),
        ],
        compiler_params=pltpu.CompilerParams(
            collective_id=0, vmem_limit_bytes=48 * 1024 * 1024),
    )(x)


# device time: 243818 ns/iter; 1.0606x vs baseline; 1.0465x over previous
import functools

import jax
import jax.numpy as jnp
from jax import lax
from jax.experimental import pallas as pl
from jax.experimental.pallas import tpu as pltpu


def kernel(x):
    m, n = x.shape
    H = m // 2
    CH = 512
    K = m // CH
    KD = H // CH

    def body(x_ref, out_ref, vshard, vload, lsems, ssem,
             ysend, yrecv, xsend, xrecv):
        my_x = lax.axis_index("x")
        my_y = lax.axis_index("y")
        ynbr = (my_x, 1 - my_y)
        xnbr = (1 - my_x, my_y)

        bar = pltpu.get_barrier_semaphore()
        for nbr in (ynbr, xnbr):
            pl.semaphore_signal(bar, inc=1, device_id=nbr,
                                device_id_type=pl.DeviceIdType.MESH)
        pl.semaphore_wait(bar, 2)

        dhalf = my_x * H
        phalf = (1 - my_x) * H

        def lofs(i):
            if i < KD:
                return dhalf + i * CH
            return phalf + (i - KD) * CH

        def start_load(i):
            slot = i % 2
            c = pltpu.make_async_copy(
                x_ref.at[pl.ds(lofs(i), CH)], vload.at[slot], lsems.at[slot])
            c.start()
            return c

        def convert(i, loads):
            if i + 1 < K:
                loads[i + 1] = start_load(i + 1)
            loads[i].wait()
            vshard[pl.ds(lofs(i), CH), :] = vload[i % 2].astype(jnp.bfloat16)

        loads = {0: start_load(0)}
        y_rdmas = []
        for i in range(KD):
            convert(i, loads)
            o = i * CH
            r = pltpu.make_async_remote_copy(
                src_ref=vshard.at[pl.ds(dhalf + o, CH)],
                dst_ref=out_ref.at[pl.ds(my_y * m + dhalf + o, CH)],
                send_sem=ysend.at[i], recv_sem=yrecv.at[i],
                device_id=ynbr, device_id_type=pl.DeviceIdType.MESH)
            r.start()
            y_rdmas.append(r)

        fw_rdmas = []
        for j in range(KD):
            o = j * CH
            y_rdmas[j].wait_recv()
            rofs = (1 - my_y) * m + dhalf + o
            fw = pltpu.make_async_remote_copy(
                src_ref=out_ref.at[pl.ds(rofs, CH)],
                dst_ref=out_ref.at[pl.ds(rofs, CH)],
                send_sem=xsend.at[j], recv_sem=xrecv.at[j],
                device_id=xnbr, device_id_type=pl.DeviceIdType.MESH)
            fw.start()
            fw_rdmas.append(fw)
            convert(KD + j, loads)

        st = pltpu.make_async_copy(
            vshard, out_ref.at[pl.ds(my_y * m, m)], ssem)
        st.start()

        for r in y_rdmas:
            r.wait_send()
        for r in fw_rdmas:
            r.wait_send()
        for r in fw_rdmas:
            r.wait_recv()
        st.wait()

        @functools.partial(pl.run_scoped,
                           second_barrier=pltpu.SemaphoreType.REGULAR)
        def _(second_barrier):
            for nbr in (ynbr, xnbr):
                pl.semaphore_signal(second_barrier, inc=1, device_id=nbr,
                                    device_id_type=pl.DeviceIdType.MESH)
            pl.semaphore_wait(second_barrier, 2)

    return pl.pallas_call(
        body,
        out_shape=jax.ShapeDtypeStruct((2 * m, n), jnp.bfloat16),
        in_specs=[pl.BlockSpec(memory_space=pltpu.MemorySpace.HBM)],
        out_specs=pl.BlockSpec(memory_space=pltpu.MemorySpace.HBM),
        scratch_shapes=[
            pltpu.VMEM((m, n), jnp.bfloat16),
            pltpu.VMEM((2, CH, n), jnp.float32),
            pltpu.SemaphoreType.DMA((2,)),
            pltpu.SemaphoreType.DMA,
            pltpu.SemaphoreType.DMA((KD,)),
            pltpu.SemaphoreType.DMA((KD,)),
            pltpu.SemaphoreType.DMA((KD,)),
            pltpu.SemaphoreType.DMA((KD,)),
        ],
        compiler_params=pltpu.CompilerParams(
            collective_id=0, vmem_limit_bytes=48 * 1024 * 1024),
    )(x)
